# initial kernel scaffold (unmeasured)
import jax
import jax.numpy as jnp
from jax import lax
from jax.experimental import pallas as pl
from jax.experimental.pallas import tpu as pltpu

M = 2048
N = 1024
C = 256
NCHUNK = M // C


def kernel(x, dest):
    me = lax.axis_index("y")
    n0 = jnp.sum((dest == 0).astype(jnp.int32))
    s = jnp.where(me == 0, M - n0, n0).astype(jnp.int32)

    perm = jnp.argsort(dest, stable=True)
    xs = x.astype(jnp.bfloat16)[perm]
    scal = jnp.reshape(s, (1,)).astype(jnp.int32)

    def body(scal_ref, xs_ref, out_ref, copy_sem, send_sems, recv_sems):
        s_ = scal_ref[0]
        me_ = lax.axis_index("y")
        mx_ = lax.axis_index("x")
        other = 1 - me_
        send_off = jnp.where(me_ == 0, M - s_, 0)
        remote_off = (M - s_) - send_off

        cp = pltpu.make_async_copy(xs_ref, out_ref, copy_sem)
        cp.start()
        cp.wait()

        bsem = pltpu.get_barrier_semaphore()
        pl.semaphore_signal(
            bsem, inc=1, device_id=(mx_, other),
            device_id_type=pl.DeviceIdType.MESH,
        )
        pl.semaphore_wait(bsem, 1)

        def mk(ci):
            st = jnp.maximum(0, jnp.minimum(ci * C, s_ - C))
            return pltpu.make_async_remote_copy(
                src_ref=xs_ref.at[pl.ds(send_off + st, C)],
                dst_ref=out_ref.at[pl.ds(remote_off + st, C)],
                send_sem=send_sems.at[ci],
                recv_sem=recv_sems.at[ci],
                device_id=(mx_, other),
                device_id_type=pl.DeviceIdType.MESH,
            )

        for ci in range(NCHUNK):
            @pl.when(ci * C < s_)
            def _():
                mk(ci).start()

        for ci in range(NCHUNK):
            @pl.when(ci * C < s_)
            def _():
                mk(ci).wait()

    return pl.pallas_call(
        body,
        out_shape=jax.ShapeDtypeStruct((M, N), jnp.bfloat16),
        in_specs=[
            pl.BlockSpec(memory_space=pltpu.SMEM),
            pl.BlockSpec(memory_space=pltpu.VMEM),
        ],
        out_specs=pl.BlockSpec(memory_space=pltpu.VMEM),
        scratch_shapes=[
            pltpu.SemaphoreType.DMA,
            pltpu.SemaphoreType.DMA((NCHUNK,)),
            pltpu.SemaphoreType.DMA((NCHUNK,)),
        ],
        compiler_params=pltpu.CompilerParams(collective_id=0),
    )(scal, xs)


# baseline (device time: 45637 ns/iter reference)
import jax
import jax.numpy as jnp
from jax import lax
from jax.experimental import pallas as pl
from jax.experimental.pallas import tpu as pltpu

M = 2048
N = 1024
C = 256
NCHUNK = M // C


def kernel(x, dest):
    me = lax.axis_index("y")
    iszero = (dest == 0).astype(jnp.int32)
    cz = jnp.cumsum(iszero)
    n0 = cz[-1]
    s = jnp.where(me == 0, M - n0, n0).astype(jnp.int32)

    co = jnp.arange(1, M + 1, dtype=jnp.int32) - cz
    pos = jnp.where(iszero == 1, cz - 1, n0 + co - 1)
    xs = (
        jnp.zeros((M, N), jnp.bfloat16)
        .at[pos]
        .set(x.astype(jnp.bfloat16), unique_indices=True, mode="drop")
    )
    xs = xs.reshape(M, 8, 128)
    scal = jnp.reshape(s, (1,)).astype(jnp.int32)

    def body(scal_ref, xs_ref, out_ref, copy_sems, send_sems, recv_sems):
        s_ = scal_ref[0]
        me_ = lax.axis_index("y")
        mx_ = lax.axis_index("x")
        other = 1 - me_
        k = M - s_
        send_off = jnp.where(me_ == 0, k, 0)
        remote_off = k - send_off
        koff = jnp.where(me_ == 0, 0, s_)

        bsem = pltpu.get_barrier_semaphore()
        pl.semaphore_signal(
            bsem, inc=1, device_id=(mx_, other),
            device_id_type=pl.DeviceIdType.MESH,
        )
        pl.semaphore_wait(bsem, 1)

        def mk(ci):
            st = jnp.maximum(0, jnp.minimum(ci * C, s_ - C))
            return pltpu.make_async_remote_copy(
                src_ref=xs_ref.at[pl.ds(send_off + st, C)],
                dst_ref=out_ref.at[pl.ds(remote_off + st, C)],
                send_sem=send_sems.at[ci],
                recv_sem=recv_sems.at[ci],
                device_id=(mx_, other),
                device_id_type=pl.DeviceIdType.MESH,
            )

        def mk_keep(ci):
            st = jnp.maximum(0, jnp.minimum(ci * C, k - C))
            return pltpu.make_async_copy(
                xs_ref.at[pl.ds(koff + st, C)],
                out_ref.at[pl.ds(koff + st, C)],
                copy_sems.at[ci],
            )

        for ci in range(NCHUNK):
            @pl.when(ci * C < s_)
            def _():
                mk(ci).start()

        for ci in range(NCHUNK):
            @pl.when(ci * C < k)
            def _():
                mk_keep(ci).start()
        for ci in range(NCHUNK):
            @pl.when(ci * C < k)
            def _():
                mk_keep(ci).wait()

        for ci in range(NCHUNK):
            @pl.when(ci * C < s_)
            def _():
                mk(ci).wait()

    out = pl.pallas_call(
        body,
        out_shape=jax.ShapeDtypeStruct((M, 8, 128), jnp.bfloat16),
        in_specs=[
            pl.BlockSpec(memory_space=pltpu.SMEM),
            pl.BlockSpec(memory_space=pltpu.VMEM),
        ],
        out_specs=pl.BlockSpec(memory_space=pltpu.VMEM),
        scratch_shapes=[
            pltpu.SemaphoreType.DMA((NCHUNK,)),
            pltpu.SemaphoreType.DMA((NCHUNK,)),
            pltpu.SemaphoreType.DMA((NCHUNK,)),
        ],
        compiler_params=pltpu.CompilerParams(collective_id=0),
    )(scal, xs)
    return out.reshape(M, N)


# device time: 39569 ns/iter; 1.1534x vs baseline; 1.1534x over previous
import jax
import jax.numpy as jnp
from jax import lax
from jax.experimental import pallas as pl
from jax.experimental.pallas import tpu as pltpu

M = 2048
N = 1024
CK = 256
NK = M // CK
C = 64
NCH = (M // 2) // C + 1


def kernel(x, dest):
    me = lax.axis_index("y")
    iszero = (dest == 0).astype(jnp.int32)
    cz = jnp.cumsum(iszero)
    n0 = cz[-1]
    s = jnp.where(me == 0, M - n0, n0).astype(jnp.int32)

    mx = lax.axis_index("x")
    co = jnp.arange(1, M + 1, dtype=jnp.int32) - cz
    pos = jnp.where(iszero == 1, cz - 1, n0 + co - 1)
    H = s // 2
    hoff_my = jnp.where(mx == 0, 0, H)
    len_my = jnp.where(mx == 0, H, s - H)
    send_rank = jnp.where(me == 0, co, cz) - 1
    is_send = dest != me
    outside = is_send & (
        (send_rank < hoff_my) | (send_rank >= hoff_my + len_my)
    )
    pos = jnp.where(outside, M, pos)
    xs = (
        jnp.zeros((M, N), jnp.bfloat16)
        .at[pos]
        .set(x.astype(jnp.bfloat16), unique_indices=True, mode="drop")
    )
    xs = xs.reshape(M, 8, 128)
    scal = jnp.reshape(s, (1,)).astype(jnp.int32)

    def body(scal_ref, xs_ref, out_ref,
             keep_sems, ysend_sems, yrecv_sems, fsend_sems, frecv_sems):
        s_ = scal_ref[0]
        me_ = lax.axis_index("y")
        mx_ = lax.axis_index("x")
        oy = 1 - me_
        ox = 1 - mx_
        k = M - s_
        send_off = jnp.where(me_ == 0, k, 0)
        ro = send_off
        remote_off = k - send_off
        koff = jnp.where(me_ == 0, 0, s_)
        H = s_ // 2
        s2 = s_ - H
        hoff_my = jnp.where(mx_ == 0, 0, H)
        len_my = jnp.where(mx_ == 0, H, s2)
        hoff_ox = jnp.where(mx_ == 0, H, 0)
        len_ox = jnp.where(mx_ == 0, s2, H)

        bsem = pltpu.get_barrier_semaphore()
        for dev in [(mx_, oy), (ox, me_)]:
            pl.semaphore_signal(
                bsem, inc=1, device_id=dev,
                device_id_type=pl.DeviceIdType.MESH,
            )
        pl.semaphore_wait(bsem, 2)

        def cst(ci, ln):
            return jnp.maximum(0, jnp.minimum(ci * C, ln - C))

        def y_rdma(ci):
            st = hoff_my + cst(ci, len_my)
            return pltpu.make_async_remote_copy(
                src_ref=xs_ref.at[pl.ds(send_off + st, C)],
                dst_ref=out_ref.at[pl.ds(remote_off + st, C)],
                send_sem=ysend_sems.at[ci],
                recv_sem=yrecv_sems.at[ci],
                device_id=(mx_, oy),
                device_id_type=pl.DeviceIdType.MESH,
            )

        def fwd_rdma(ci):
            st = ro + hoff_my + cst(ci, len_my)
            return pltpu.make_async_remote_copy(
                src_ref=out_ref.at[pl.ds(st, C)],
                dst_ref=out_ref.at[pl.ds(st, C)],
                send_sem=fsend_sems.at[ci],
                recv_sem=frecv_sems.at[ci],
                device_id=(ox, me_),
                device_id_type=pl.DeviceIdType.MESH,
            )

        def frecv_rdma(ci):
            st = ro + hoff_ox + cst(ci, len_ox)
            return pltpu.make_async_remote_copy(
                src_ref=out_ref.at[pl.ds(st, C)],
                dst_ref=out_ref.at[pl.ds(st, C)],
                send_sem=fsend_sems.at[ci],
                recv_sem=frecv_sems.at[ci],
                device_id=(ox, me_),
                device_id_type=pl.DeviceIdType.MESH,
            )

        def keep_dma(ci):
            st = koff + jnp.maximum(0, jnp.minimum(ci * CK, k - CK))
            return pltpu.make_async_copy(
                xs_ref.at[pl.ds(st, CK)],
                out_ref.at[pl.ds(st, CK)],
                keep_sems.at[ci],
            )

        for ci in range(NCH):
            @pl.when(ci * C < len_my)
            def _():
                y_rdma(ci).start()

        for ci in range(NK):
            @pl.when(ci * CK < k)
            def _():
                keep_dma(ci).start()

        for ci in range(NCH):
            @pl.when(ci * C < len_my)
            def _():
                y_rdma(ci).wait_recv()
                fwd_rdma(ci).start()

        for ci in range(NCH):
            @pl.when(ci * C < len_ox)
            def _():
                frecv_rdma(ci).wait_recv()

        for ci in range(NK):
            @pl.when(ci * CK < k)
            def _():
                keep_dma(ci).wait()
        for ci in range(NCH):
            @pl.when(ci * C < len_my)
            def _():
                y_rdma(ci).wait_send()
                fwd_rdma(ci).wait_send()

    out = pl.pallas_call(
        body,
        out_shape=jax.ShapeDtypeStruct((M, 8, 128), jnp.bfloat16),
        in_specs=[
            pl.BlockSpec(memory_space=pltpu.SMEM),
            pl.BlockSpec(memory_space=pltpu.VMEM),
        ],
        out_specs=pl.BlockSpec(memory_space=pltpu.VMEM),
        scratch_shapes=[
            pltpu.SemaphoreType.DMA((NK,)),
            pltpu.SemaphoreType.DMA((NCH,)),
            pltpu.SemaphoreType.DMA((NCH,)),
            pltpu.SemaphoreType.DMA((NCH,)),
            pltpu.SemaphoreType.DMA((NCH,)),
        ],
        compiler_params=pltpu.CompilerParams(collective_id=0),
    )(scal, xs)
    return out.reshape(M, N)


# device time: 38650 ns/iter; 1.1808x vs baseline; 1.0238x over previous
import jax
import jax.numpy as jnp
from jax import lax
from jax.experimental import pallas as pl
from jax.experimental.pallas import tpu as pltpu

M = 2048
N = 1024
CK = 256
NK = M // CK
C = 64
NCH = (M // 2) // C + 1


def kernel(x, dest):
    me = lax.axis_index("y")
    iszero = (dest == 0).astype(jnp.int32)
    cz = jnp.cumsum(iszero)
    n0 = cz[-1]
    s = jnp.where(me == 0, M - n0, n0).astype(jnp.int32)

    co = jnp.arange(1, M + 1, dtype=jnp.int32) - cz
    pos = jnp.where(iszero == 1, cz - 1, n0 + co - 1)
    xs = (
        jnp.zeros((M, N), jnp.bfloat16)
        .at[pos]
        .set(x.astype(jnp.bfloat16), unique_indices=True, mode="drop")
    )
    xs = xs.reshape(M, 8, 128)
    scal = jnp.reshape(s, (1,)).astype(jnp.int32)

    def body(scal_ref, xs_ref, out_ref,
             keep_sems, ysend_sems, yrecv_sems, fsend_sems, frecv_sems):
        s_ = scal_ref[0]
        me_ = lax.axis_index("y")
        mx_ = lax.axis_index("x")
        oy = 1 - me_
        ox = 1 - mx_
        k = M - s_
        send_off = jnp.where(me_ == 0, k, 0)
        ro = send_off
        remote_off = k - send_off
        koff = jnp.where(me_ == 0, 0, s_)
        H = s_ // 2
        s2 = s_ - H
        hoff_my = jnp.where(mx_ == 0, 0, H)
        len_my = jnp.where(mx_ == 0, H, s2)
        hoff_ox = jnp.where(mx_ == 0, H, 0)
        len_ox = jnp.where(mx_ == 0, s2, H)

        bsem = pltpu.get_barrier_semaphore()
        for dev in [(mx_, oy), (ox, me_)]:
            pl.semaphore_signal(
                bsem, inc=1, device_id=dev,
                device_id_type=pl.DeviceIdType.MESH,
            )
        pl.semaphore_wait(bsem, 2)

        def cst(ci, ln):
            return jnp.maximum(0, jnp.minimum(ci * C, ln - C))

        def y_rdma(ci):
            st = hoff_my + cst(ci, len_my)
            return pltpu.make_async_remote_copy(
                src_ref=xs_ref.at[pl.ds(send_off + st, C)],
                dst_ref=out_ref.at[pl.ds(remote_off + st, C)],
                send_sem=ysend_sems.at[ci],
                recv_sem=yrecv_sems.at[ci],
                device_id=(mx_, oy),
                device_id_type=pl.DeviceIdType.MESH,
            )

        def fwd_rdma(ci):
            st = ro + hoff_my + cst(ci, len_my)
            return pltpu.make_async_remote_copy(
                src_ref=out_ref.at[pl.ds(st, C)],
                dst_ref=out_ref.at[pl.ds(st, C)],
                send_sem=fsend_sems.at[ci],
                recv_sem=frecv_sems.at[ci],
                device_id=(ox, me_),
                device_id_type=pl.DeviceIdType.MESH,
            )

        def frecv_rdma(ci):
            st = ro + hoff_ox + cst(ci, len_ox)
            return pltpu.make_async_remote_copy(
                src_ref=out_ref.at[pl.ds(st, C)],
                dst_ref=out_ref.at[pl.ds(st, C)],
                send_sem=fsend_sems.at[ci],
                recv_sem=frecv_sems.at[ci],
                device_id=(ox, me_),
                device_id_type=pl.DeviceIdType.MESH,
            )

        def keep_dma(ci):
            st = koff + jnp.maximum(0, jnp.minimum(ci * CK, k - CK))
            return pltpu.make_async_copy(
                xs_ref.at[pl.ds(st, CK)],
                out_ref.at[pl.ds(st, CK)],
                keep_sems.at[ci],
            )

        for ci in range(NCH):
            @pl.when(ci * C < len_my)
            def _():
                y_rdma(ci).start()

        for ci in range(NK):
            @pl.when(ci * CK < k)
            def _():
                keep_dma(ci).start()

        for ci in range(NCH):
            @pl.when(ci * C < len_my)
            def _():
                y_rdma(ci).wait_recv()
                fwd_rdma(ci).start()

        for ci in range(NCH):
            @pl.when(ci * C < len_ox)
            def _():
                frecv_rdma(ci).wait_recv()

        for ci in range(NK):
            @pl.when(ci * CK < k)
            def _():
                keep_dma(ci).wait()
        for ci in range(NCH):
            @pl.when(ci * C < len_my)
            def _():
                y_rdma(ci).wait_send()
                fwd_rdma(ci).wait_send()

    out = pl.pallas_call(
        body,
        out_shape=jax.ShapeDtypeStruct((M, 8, 128), jnp.bfloat16),
        in_specs=[
            pl.BlockSpec(memory_space=pltpu.SMEM),
            pl.BlockSpec(memory_space=pltpu.VMEM),
        ],
        out_specs=pl.BlockSpec(memory_space=pltpu.VMEM),
        scratch_shapes=[
            pltpu.SemaphoreType.DMA((NK,)),
            pltpu.SemaphoreType.DMA((NCH,)),
            pltpu.SemaphoreType.DMA((NCH,)),
            pltpu.SemaphoreType.DMA((NCH,)),
            pltpu.SemaphoreType.DMA((NCH,)),
        ],
        compiler_params=pltpu.CompilerParams(collective_id=0),
    )(scal, xs)
    return out.reshape(M, N)
